# baseline (speedup 1.0000x reference)
import functools

import jax
import jax.numpy as jnp
from jax import lax
from jax.experimental import pallas as pl
from jax.experimental.pallas import tpu as pltpu
from jax.experimental.pallas import tpu_sc as plsc

N_ROWS = 8192
N_CODES = 8192
DIM = 256

R_BLK = 512
C_BLK = 1024
I_BLKS = N_ROWS // R_BLK
J_BLKS = N_CODES // C_BLK
HCHUNK = 2048


def _vq_body(x_ref, e_ref, a2_ref, b2_ref, w_ref,
             idx_ref, com_ref, per_ref, act_ref,
             ab0_ref, ab1_ref, rmin_ref, ridx_ref, lprev_ref, cnt_ref):
    i = pl.program_id(0)
    j = pl.program_id(1)
    par = j % 2

    @pl.when((i < I_BLKS) & (j < J_BLKS) & (par == 0))
    def _mm0():
        ab0_ref[...] = lax.dot_general(
            x_ref[...], e_ref[...], (((1,), (1,)), ((), ())),
            preferred_element_type=jnp.float32)

    @pl.when((i < I_BLKS) & (j < J_BLKS) & (par == 1))
    def _mm1():
        ab1_ref[...] = lax.dot_general(
            x_ref[...], e_ref[...], (((1,), (1,)), ((), ())),
            preferred_element_type=jnp.float32)

    def _epilogue(ab_ref):
        jj = j - 1
        ab2 = ab_ref[...]
        d2 = (a2_ref[...] + ab2) + b2_ref[:, pl.ds(jj * C_BLK, C_BLK)]
        m2 = jnp.min(d2, axis=1, keepdims=True)
        m2c = jnp.maximum(m2, 0.0)
        m = jnp.sqrt(m2c)
        hi = m2c
        bits = lax.bitcast_convert_type(m2c, jnp.int32)
        for jp in range(1, 5):
            cand = lax.bitcast_convert_type(bits + jp, jnp.float32)
            hi = jnp.where(jnp.sqrt(cand) == m, cand, hi)
        col = lax.broadcasted_iota(jnp.int32, (R_BLK, C_BLK), 1) + jj * C_BLK
        lidx = jnp.min(jnp.where(d2 <= hi, col, N_CODES),
                       axis=1, keepdims=True)
        prev_m = jnp.where(jj == 0, jnp.inf, rmin_ref[...])
        prev_i = ridx_ref[...]
        better = m < prev_m
        rmin_ref[...] = jnp.where(better, m, prev_m)
        ridx_ref[...] = jnp.where(better, lidx, prev_i)

    @pl.when((i < I_BLKS) & (j >= 1) & (par == 1))
    def _ep0():
        _epilogue(ab0_ref)

    @pl.when((i < I_BLKS) & (j >= 1) & (par == 0))
    def _ep1():
        _epilogue(ab1_ref)

    @pl.when((i < I_BLKS) & (j == J_BLKS))
    def _finalize_rows():
        new_i = ridx_ref[...]
        new_m = rmin_ref[...]
        idx_ref[...] = new_i
        lprev_ref[...] = new_i
        bsum = jnp.sum(new_m * new_m)
        prev = jnp.where(i == 0, 0.0, com_ref[0, 0])
        tot = prev + bsum
        com_ref[0, 0] = jnp.where(i == I_BLKS - 1,
                                  tot * (1.0 / (N_ROWS * DIM)), tot)

    @pl.when((i >= 1) & (j < N_CODES // HCHUNK))
    def _hist_chunk():
        lp = lprev_ref[...]
        bins = lax.broadcasted_iota(jnp.int32, (R_BLK, HCHUNK), 1) + j * HCHUNK
        eqf = (lp == bins).astype(jnp.float32)
        csum = jnp.sum(eqf, axis=0, keepdims=True)
        prev_c = jnp.where(i == 1, 0.0, cnt_ref[0:1, pl.ds(j * HCHUNK, HCHUNK)])
        cnt_ref[0:1, pl.ds(j * HCHUNK, HCHUNK)] = prev_c + csum

    @pl.when((i == I_BLKS) & (j == N_CODES // HCHUNK))
    def _finalize_scalars():
        p = cnt_ref[...] * (1.0 / N_ROWS)
        s = jnp.sum(p * jnp.log(p + 1e-10))
        per_ref[0, 0] = jnp.exp(-s)

    @pl.when((i == 0) & (j == 0))
    def _active():
        act_ref[0, 0] = jnp.sum((w_ref[...] >= 0.01).astype(jnp.int32))


def _vq_core(xf, emb, a2, b2, wrow):
    return pl.pallas_call(
        _vq_body,
        grid=(I_BLKS + 1, J_BLKS + 1),
        in_specs=[
            pl.BlockSpec((R_BLK, DIM), lambda i, j: (jnp.minimum(i, I_BLKS - 1), 0)),
            pl.BlockSpec((C_BLK, DIM), lambda i, j: (jnp.minimum(j, J_BLKS - 1), 0)),
            pl.BlockSpec((R_BLK, 1), lambda i, j: (jnp.minimum(i, I_BLKS - 1), 0)),
            pl.BlockSpec((1, N_CODES), lambda i, j: (0, 0)),
            pl.BlockSpec((1, N_CODES), lambda i, j: (0, 0)),
        ],
        out_specs=[
            pl.BlockSpec((R_BLK, 1), lambda i, j: (jnp.minimum(i, I_BLKS - 1), 0)),
            pl.BlockSpec(memory_space=pltpu.SMEM),
            pl.BlockSpec(memory_space=pltpu.SMEM),
            pl.BlockSpec(memory_space=pltpu.SMEM),
        ],
        out_shape=[
            jax.ShapeDtypeStruct((N_ROWS, 1), jnp.int32),
            jax.ShapeDtypeStruct((1, 1), jnp.float32),
            jax.ShapeDtypeStruct((1, 1), jnp.float32),
            jax.ShapeDtypeStruct((1, 1), jnp.int32),
        ],
        scratch_shapes=[
            pltpu.VMEM((R_BLK, C_BLK), jnp.float32),
            pltpu.VMEM((R_BLK, C_BLK), jnp.float32),
            pltpu.VMEM((R_BLK, 1), jnp.float32),
            pltpu.VMEM((R_BLK, 1), jnp.int32),
            pltpu.VMEM((R_BLK, 1), jnp.int32),
            pltpu.VMEM((1, N_CODES), jnp.float32),
        ],
    )(xf, emb, a2, b2, wrow)


def _gather_rows(emb, idx):
    info = plsc.get_sparse_core_info()
    nw = info.num_cores * info.num_subcores
    bpw = N_ROWS // nw

    @functools.partial(
        pl.kernel,
        out_type=jax.ShapeDtypeStruct((N_ROWS, DIM), jnp.float32),
        mesh=plsc.VectorSubcoreMesh(core_axis_name="c", subcore_axis_name="s"),
        scratch_types=[
            pltpu.VMEM((bpw,), jnp.int32),
            pltpu.VMEM((bpw, DIM), jnp.float32),
            pltpu.SemaphoreType.DMA,
        ],
    )
    def k(emb_hbm, idx_hbm, out_hbm, idx_v, rows_v, sem):
        wid = lax.axis_index("s") * info.num_cores + lax.axis_index("c")
        base = wid * bpw
        pltpu.sync_copy(idx_hbm.at[pl.ds(base, bpw)], idx_v)
        pltpu.async_copy(emb_hbm.at[idx_v], rows_v, sem).wait()
        pltpu.sync_copy(rows_v, out_hbm.at[pl.ds(base, bpw)])

    return k(emb, idx)


def kernel(inputs, emb_weight, weight):
    x = jnp.transpose(inputs, (0, 2, 3, 1))
    input_shape = x.shape
    xf = x.reshape(N_ROWS, DIM)
    a2 = jnp.sum(xf * xf, axis=1, keepdims=True)
    b2 = jnp.sum(emb_weight * emb_weight, axis=1)[None, :]
    wrow = weight.reshape(1, N_CODES)
    xm2 = xf * (-2.0)

    idx2d, com, per, act = _vq_core(xm2, emb_weight, a2, b2, wrow)
    indices = idx2d[:, 0]

    q = _gather_rows(emb_weight, indices)
    quantized = jnp.transpose(q.reshape(input_shape), (0, 3, 1, 2))

    return (quantized, com[0, 0], per[0, 0], act[0, 0], indices)
